# Initial kernel scaffold; baseline (speedup 1.0000x reference)
import jax
import jax.numpy as jnp
from jax.experimental import pallas as pl


def kernel(x, edge_index, batch, W1, as1, ad1, b1, W2, as2, ad2, b2, Wfc, bfc):
    raise NotImplementedError("write your pallas kernel here")


# speedup vs baseline: 14.5858x; 14.5858x over previous
import functools

import jax
import jax.numpy as jnp
from jax import lax
from jax.experimental import pallas as pl
from jax.experimental.pallas import tpu as pltpu
from jax.experimental.pallas import tpu_sc as plsc

_N = 10000
_E = 320000
_H = 128
_G = 64
_NW = 32
_K = 128
_RPT = 80
_CHUNK = 4
_EPAD = _NW * _RPT * _K
_ACC_ROWS = 10016


def _dense_body(x_ref, w_ref, a_ref, h_ref, av_ref):
    h = jnp.dot(x_ref[...], w_ref[...], preferred_element_type=jnp.float32)
    h_ref[...] = h
    av_ref[...] = jnp.dot(h, a_ref[...], preferred_element_type=jnp.float32)


def _dense(x, W, A):
    return pl.pallas_call(
        _dense_body,
        out_shape=[
            jax.ShapeDtypeStruct((_N, _H), jnp.float32),
            jax.ShapeDtypeStruct((_N, 2), jnp.float32),
        ],
    )(x, W, A)


def _mid_body(num_ref, den_ref, b_ref, w_ref, a_ref, h_ref, av_ref):
    num = num_ref[0] + num_ref[1]
    den = den_ref[0, :, 0:1] + den_ref[1, :, 0:1]
    xr = jnp.maximum(num / (den + 1e-16) + b_ref[...], 0.0)
    h = jnp.dot(xr, w_ref[...], preferred_element_type=jnp.float32)
    h_ref[...] = h
    av_ref[...] = jnp.dot(h, a_ref[...], preferred_element_type=jnp.float32)


def _mid(accn, accd, b, W, A):
    return pl.pallas_call(
        _mid_body,
        out_shape=[
            jax.ShapeDtypeStruct((_N, _H), jnp.float32),
            jax.ShapeDtypeStruct((_N, 2), jnp.float32),
        ],
    )(accn, accd, b, W, A)


def _final_body(num_ref, den_ref, b_ref, batch_ref, wfc_ref, bfc_ref, out_ref):
    num = num_ref[0] + num_ref[1]
    den = den_ref[0, :, 0:1] + den_ref[1, :, 0:1]
    hf = num / (den + 1e-16) + b_ref[...]
    bt = batch_ref[...]
    rows = []
    for g in range(_G):
        m = bt == g
        rows.append(jnp.max(jnp.where(m, hf, -jnp.inf), axis=0, keepdims=True))
    pooled = jnp.concatenate(rows, axis=0)
    pooled = jnp.where(jnp.isfinite(pooled), pooled, 0.0)
    logits = jnp.dot(pooled, wfc_ref[...],
                     preferred_element_type=jnp.float32) + bfc_ref[...]
    mx = jnp.max(logits, axis=1, keepdims=True)
    s = logits - mx
    lse = jnp.log(jnp.sum(jnp.exp(s), axis=1, keepdims=True))
    out_ref[...] = s - lse


def _final(accn, accd, b, batch2d, Wfc, bfc):
    return pl.pallas_call(
        _final_body,
        out_shape=jax.ShapeDtypeStruct((_G, Wfc.shape[1]), jnp.float32),
    )(accn, accd, b, batch2d, Wfc, bfc)


def _edge_body(h_hbm, avs_hbm, avd_hbm, src_hbm, dst_hbm,
               outn_hbm, outd_hbm,
               avs_v, avd_v, src_v, dst_v, rbuf, wbuf, accn, accd, sem):
    cid = lax.axis_index("c")
    sid = lax.axis_index("s")
    wid = sid * 2 + cid

    zero16 = jnp.zeros((16,), jnp.float32)

    def zrow(i, _):
        for j in range(8):
            rbuf[i, pl.ds(16 * j, 16)] = zero16
        wbuf[i, pl.ds(0, 16)] = zero16
        return 0
    lax.fori_loop(0, _K, zrow, 0)

    zbase = sid * 626
    for t in range(4):
        pltpu.sync_copy(rbuf, accn.at[pl.ds(zbase + 128 * t, 128)])
        pltpu.sync_copy(wbuf, accd.at[pl.ds(zbase + 128 * t, 128)])
    pltpu.sync_copy(rbuf.at[pl.ds(0, 114)], accn.at[pl.ds(zbase + 512, 114)])
    pltpu.sync_copy(wbuf.at[pl.ds(0, 114)], accd.at[pl.ds(zbase + 512, 114)])

    pltpu.sync_copy(avs_hbm, avs_v)
    pltpu.sync_copy(avd_hbm, avd_v)
    plsc.subcore_barrier()

    iota16 = lax.iota(jnp.int32, 16)

    def chunk_body(c, _):
        ibase = wid * _RPT + c * _CHUNK
        pltpu.sync_copy(src_hbm.at[pl.ds(ibase, _CHUNK)], src_v)
        pltpu.sync_copy(dst_hbm.at[pl.ds(ibase, _CHUNK)], dst_v)

        def batch_body(b, _):
            pltpu.async_copy(h_hbm.at[src_v.at[b]], rbuf, sem).wait()
            for j in range(8):
                sl = pl.ds(16 * j, 16)
                isrc = src_v[b, sl]
                idst = jnp.minimum(dst_v[b, sl], _N - 1)
                e = (plsc.load_gather(avs_v, [isrc])
                     + plsc.load_gather(avd_v, [idst]))
                e = jnp.where(e >= 0.0, e, 0.2 * e)
                wv = jnp.exp(e)
                for i in range(16):
                    s = wv[i]
                    row = 16 * j + i
                    for d in range(8):
                        dsl = pl.ds(16 * d, 16)
                        rbuf[row, dsl] = rbuf[row, dsl] * s
                    wbuf[row, pl.ds(0, 16)] = jnp.where(iota16 == 0, s, 0.0)

            pltpu.sync_copy(rbuf, accn.at[dst_v.at[b]], add=True)
            pltpu.sync_copy(wbuf, accd.at[dst_v.at[b]], add=True)
            return 0
        lax.fori_loop(0, _CHUNK, batch_body, 0)
        return 0
    lax.fori_loop(0, _RPT // _CHUNK, chunk_body, 0)

    plsc.subcore_barrier()
    obase = sid * 625
    pltpu.sync_copy(accn.at[pl.ds(obase, 625)],
                    outn_hbm.at[cid].at[pl.ds(obase, 625)])
    pltpu.sync_copy(accd.at[pl.ds(obase, 625)],
                    outd_hbm.at[cid].at[pl.ds(obase, 625)])


def _edge_pass(h, avs, avd, src2d, dst2d):
    mesh = plsc.VectorSubcoreMesh(core_axis_name="c", subcore_axis_name="s")
    f = pl.kernel(
        _edge_body,
        out_type=[
            jax.ShapeDtypeStruct((2, _N, 128), jnp.float32),
            jax.ShapeDtypeStruct((2, _N, 16), jnp.float32),
        ],
        mesh=mesh,
        compiler_params=pltpu.CompilerParams(use_tc_tiling_on_sc=False,
                                             needs_layout_passes=False),
        scratch_types=[
            pltpu.VMEM((_N,), jnp.float32),
            pltpu.VMEM((_N,), jnp.float32),
            pltpu.VMEM((_CHUNK, _K), jnp.int32),
            pltpu.VMEM((_CHUNK, _K), jnp.int32),
            pltpu.VMEM((_K, 128), jnp.float32),
            pltpu.VMEM((_K, 16), jnp.float32),
            pltpu.VMEM_SHARED((_ACC_ROWS, 128), jnp.float32),
            pltpu.VMEM_SHARED((_ACC_ROWS, 16), jnp.float32),
            pltpu.SemaphoreType.DMA,
        ],
    )
    return f(h, avs, avd, src2d, dst2d)


def kernel(x, edge_index, batch, W1, as1, ad1, b1, W2, as2, ad2, b2, Wfc, bfc):
    src = edge_index[0]
    dst = edge_index[1]
    pad = _EPAD - _E
    srcp = jnp.concatenate(
        [src, jnp.zeros((pad,), jnp.int32)]).reshape(_NW * _RPT, _K)
    dstp = jnp.concatenate(
        [dst, jnp.full((pad,), _N, jnp.int32)]).reshape(_NW * _RPT, _K)

    A1 = jnp.stack([as1, ad1], axis=1)
    A2 = jnp.stack([as2, ad2], axis=1)

    h1, av1 = _dense(x, W1, A1)
    n1, d1 = _edge_pass(h1, av1[:, 0], av1[:, 1], srcp, dstp)
    h2, av2 = _mid(n1, d1, b1.reshape(1, _H), W2, A2)
    n2, d2 = _edge_pass(h2, av2[:, 0], av2[:, 1], srcp, dstp)
    return _final(n2, d2, b2.reshape(1, _H), batch.reshape(_N, 1), Wfc, bfc)


# speedup vs baseline: 14.6337x; 1.0033x over previous
import functools

import jax
import jax.numpy as jnp
from jax import lax
from jax.experimental import pallas as pl
from jax.experimental.pallas import tpu as pltpu
from jax.experimental.pallas import tpu_sc as plsc

_N = 10000
_E = 320000
_H = 128
_G = 64
_NW = 32
_K = 64
_CHUNK = 8
_NCH = 20
_RPB = _NCH * _CHUNK
_EPAD = _NW * _RPB * _K
_ACC_ROWS = 10016


def _dense_body(x_ref, w_ref, a_ref, h_ref, av_ref):
    h = jnp.dot(x_ref[...], w_ref[...], preferred_element_type=jnp.float32)
    h_ref[...] = h
    av_ref[...] = jnp.dot(h, a_ref[...], preferred_element_type=jnp.float32)


def _dense(x, W, A):
    return pl.pallas_call(
        _dense_body,
        out_shape=[
            jax.ShapeDtypeStruct((_N, _H), jnp.float32),
            jax.ShapeDtypeStruct((_N, 2), jnp.float32),
        ],
    )(x, W, A)


def _mid_body(num_ref, den_ref, b_ref, w_ref, a_ref, h_ref, av_ref):
    num = num_ref[0] + num_ref[1]
    den = den_ref[0, :, 0:1] + den_ref[1, :, 0:1]
    xr = jnp.maximum(num / (den + 1e-16) + b_ref[...], 0.0)
    h = jnp.dot(xr, w_ref[...], preferred_element_type=jnp.float32)
    h_ref[...] = h
    av_ref[...] = jnp.dot(h, a_ref[...], preferred_element_type=jnp.float32)


def _mid(accn, accd, b, W, A):
    return pl.pallas_call(
        _mid_body,
        out_shape=[
            jax.ShapeDtypeStruct((_N, _H), jnp.float32),
            jax.ShapeDtypeStruct((_N, 2), jnp.float32),
        ],
    )(accn, accd, b, W, A)


def _final_body(num_ref, den_ref, b_ref, batch_ref, wfc_ref, bfc_ref, out_ref):
    num = num_ref[0] + num_ref[1]
    den = den_ref[0, :, 0:1] + den_ref[1, :, 0:1]
    hf = num / (den + 1e-16) + b_ref[...]
    bt = batch_ref[...]
    rows = []
    for g in range(_G):
        m = bt == g
        rows.append(jnp.max(jnp.where(m, hf, -jnp.inf), axis=0, keepdims=True))
    pooled = jnp.concatenate(rows, axis=0)
    pooled = jnp.where(jnp.isfinite(pooled), pooled, 0.0)
    logits = jnp.dot(pooled, wfc_ref[...],
                     preferred_element_type=jnp.float32) + bfc_ref[...]
    mx = jnp.max(logits, axis=1, keepdims=True)
    s = logits - mx
    lse = jnp.log(jnp.sum(jnp.exp(s), axis=1, keepdims=True))
    out_ref[...] = s - lse


def _final(accn, accd, b, batch2d, Wfc, bfc):
    return pl.pallas_call(
        _final_body,
        out_shape=jax.ShapeDtypeStruct((_G, Wfc.shape[1]), jnp.float32),
    )(accn, accd, b, batch2d, Wfc, bfc)


def _edge_body(h_hbm, avs_hbm, avd_hbm, src_hbm, dst_hbm,
               outn_hbm, outd_hbm,
               avs_v, avd_v, src_v, dst_v, rbuf0, rbuf1, wbuf0, wbuf1,
               accn, accd, gsem0, gsem1):
    cid = lax.axis_index("c")
    sid = lax.axis_index("s")
    wid = sid * 2 + cid

    zero16 = jnp.zeros((16,), jnp.float32)

    def zrow(i, _):
        for j in range(8):
            rbuf0[i, pl.ds(16 * j, 16)] = zero16
        wbuf0[i, pl.ds(0, 16)] = zero16
        return 0
    lax.fori_loop(0, _K, zrow, 0)

    zbase = sid * 626
    for t in range(9):
        pltpu.sync_copy(rbuf0, accn.at[pl.ds(zbase + _K * t, _K)])
        pltpu.sync_copy(wbuf0, accd.at[pl.ds(zbase + _K * t, _K)])
    pltpu.sync_copy(rbuf0.at[pl.ds(0, 50)], accn.at[pl.ds(zbase + 576, 50)])
    pltpu.sync_copy(wbuf0.at[pl.ds(0, 50)], accd.at[pl.ds(zbase + 576, 50)])

    pltpu.sync_copy(avs_hbm, avs_v)
    pltpu.sync_copy(avd_hbm, avd_v)
    plsc.subcore_barrier()

    iota16 = lax.iota(jnp.int32, 16)
    rbufs = (rbuf0, rbuf1)
    wbufs = (wbuf0, wbuf1)
    gsems = (gsem0, gsem1)

    def chunk_body(c, _):
        ibase = wid * _RPB + c * _CHUNK
        pltpu.sync_copy(src_hbm.at[pl.ds(ibase, _CHUNK)], src_v)
        pltpu.sync_copy(dst_hbm.at[pl.ds(ibase, _CHUNK)], dst_v)

        pltpu.async_copy(h_hbm.at[src_v.at[0]], rbufs[0], gsem0).wait()
        for b in range(_CHUNK):
            p = b & 1
            q = 1 - p
            rb, wb = rbufs[p], wbufs[p]
            pend = None
            if b + 1 < _CHUNK:
                pend = pltpu.async_copy(
                    h_hbm.at[src_v.at[b + 1]], rbufs[q], gsems[q])
            for j in range(_K // 16):
                sl = pl.ds(16 * j, 16)
                isrc = src_v[b, sl]
                idst = jnp.minimum(dst_v[b, sl], _N - 1)
                e = (plsc.load_gather(avs_v, [isrc])
                     + plsc.load_gather(avd_v, [idst]))
                e = jnp.where(e >= 0.0, e, 0.2 * e)
                wv = jnp.exp(e)
                for i in range(16):
                    s = wv[i]
                    row = 16 * j + i
                    for d in range(8):
                        dsl = pl.ds(16 * d, 16)
                        rb[row, dsl] = rb[row, dsl] * s
                    wb[row, pl.ds(0, 16)] = jnp.where(iota16 == 0, s, 0.0)

            if pend is not None:
                pend.wait()

            pltpu.sync_copy(rb, accn.at[dst_v.at[b]], add=True)
            pltpu.sync_copy(wb, accd.at[dst_v.at[b]], add=True)
        return 0
    lax.fori_loop(0, _NCH, chunk_body, 0)

    plsc.subcore_barrier()
    obase = sid * 625
    pltpu.sync_copy(accn.at[pl.ds(obase, 625)],
                    outn_hbm.at[cid].at[pl.ds(obase, 625)])
    pltpu.sync_copy(accd.at[pl.ds(obase, 625)],
                    outd_hbm.at[cid].at[pl.ds(obase, 625)])


def _edge_pass(h, avs, avd, src2d, dst2d):
    mesh = plsc.VectorSubcoreMesh(core_axis_name="c", subcore_axis_name="s")
    f = pl.kernel(
        _edge_body,
        out_type=[
            jax.ShapeDtypeStruct((2, _N, 128), jnp.float32),
            jax.ShapeDtypeStruct((2, _N, 16), jnp.float32),
        ],
        mesh=mesh,
        compiler_params=pltpu.CompilerParams(use_tc_tiling_on_sc=False,
                                             needs_layout_passes=False),
        scratch_types=[
            pltpu.VMEM((_N,), jnp.float32),
            pltpu.VMEM((_N,), jnp.float32),
            pltpu.VMEM((_CHUNK, _K), jnp.int32),
            pltpu.VMEM((_CHUNK, _K), jnp.int32),
            pltpu.VMEM((_K, 128), jnp.float32),
            pltpu.VMEM((_K, 128), jnp.float32),
            pltpu.VMEM((_K, 16), jnp.float32),
            pltpu.VMEM((_K, 16), jnp.float32),
            pltpu.VMEM_SHARED((_ACC_ROWS, 128), jnp.float32),
            pltpu.VMEM_SHARED((_ACC_ROWS, 16), jnp.float32),
            pltpu.SemaphoreType.DMA,
            pltpu.SemaphoreType.DMA,
        ],
    )
    return f(h, avs, avd, src2d, dst2d)


def kernel(x, edge_index, batch, W1, as1, ad1, b1, W2, as2, ad2, b2, Wfc, bfc):
    src = edge_index[0]
    dst = edge_index[1]
    pad = _EPAD - _E
    srcp = jnp.concatenate(
        [src, jnp.zeros((pad,), jnp.int32)]).reshape(_NW * _RPB, _K)
    dstp = jnp.concatenate(
        [dst, jnp.full((pad,), _N, jnp.int32)]).reshape(_NW * _RPB, _K)

    A1 = jnp.stack([as1, ad1], axis=1)
    A2 = jnp.stack([as2, ad2], axis=1)

    h1, av1 = _dense(x, W1, A1)
    n1, d1 = _edge_pass(h1, av1[:, 0], av1[:, 1], srcp, dstp)
    h2, av2 = _mid(n1, d1, b1.reshape(1, _H), W2, A2)
    n2, d2 = _edge_pass(h2, av2[:, 0], av2[:, 1], srcp, dstp)
    return _final(n2, d2, b2.reshape(1, _H), batch.reshape(_N, 1), Wfc, bfc)


# speedup vs baseline: 14.6593x; 1.0017x over previous
import functools

import jax
import jax.numpy as jnp
from jax import lax
from jax.experimental import pallas as pl
from jax.experimental.pallas import tpu as pltpu
from jax.experimental.pallas import tpu_sc as plsc

_N = 10000
_E = 320000
_H = 128
_G = 64
_ACCW = 144
_NW = 32
_K = 64
_CHUNK = 8
_NCH = 20
_RPB = _NCH * _CHUNK
_EPAD = _NW * _RPB * _K
_ACC_ROWS = 10016


def _dense_body(x_ref, w_ref, a_ref, h_ref, av_ref):
    h = jnp.dot(x_ref[...], w_ref[...], preferred_element_type=jnp.float32)
    h_ref[...] = h
    av_ref[...] = jnp.dot(h, a_ref[...], preferred_element_type=jnp.float32)


def _dense(x, W, A):
    return pl.pallas_call(
        _dense_body,
        out_shape=[
            jax.ShapeDtypeStruct((_N, _H), jnp.float32),
            jax.ShapeDtypeStruct((_N, 2), jnp.float32),
        ],
    )(x, W, A)


def _mid_body(acc_ref, b_ref, w_ref, a_ref, h_ref, av_ref):
    num = acc_ref[0, :, :128] + acc_ref[1, :, :128]
    den = acc_ref[0, :, 128:129] + acc_ref[1, :, 128:129]
    xr = jnp.maximum(num / (den + 1e-16) + b_ref[...], 0.0)
    h = jnp.dot(xr, w_ref[...], preferred_element_type=jnp.float32)
    h_ref[...] = h
    av_ref[...] = jnp.dot(h, a_ref[...], preferred_element_type=jnp.float32)


def _mid(acc, b, W, A):
    return pl.pallas_call(
        _mid_body,
        out_shape=[
            jax.ShapeDtypeStruct((_N, _H), jnp.float32),
            jax.ShapeDtypeStruct((_N, 2), jnp.float32),
        ],
    )(acc, b, W, A)


def _final_body(acc_ref, b_ref, batch_ref, wfc_ref, bfc_ref, out_ref):
    num = acc_ref[0, :, :128] + acc_ref[1, :, :128]
    den = acc_ref[0, :, 128:129] + acc_ref[1, :, 128:129]
    hf = num / (den + 1e-16) + b_ref[...]
    bt = batch_ref[...]
    rows = []
    for g in range(_G):
        m = bt == g
        rows.append(jnp.max(jnp.where(m, hf, -jnp.inf), axis=0, keepdims=True))
    pooled = jnp.concatenate(rows, axis=0)
    pooled = jnp.where(jnp.isfinite(pooled), pooled, 0.0)
    logits = jnp.dot(pooled, wfc_ref[...],
                     preferred_element_type=jnp.float32) + bfc_ref[...]
    mx = jnp.max(logits, axis=1, keepdims=True)
    s = logits - mx
    lse = jnp.log(jnp.sum(jnp.exp(s), axis=1, keepdims=True))
    out_ref[...] = s - lse


def _final(acc, b, batch2d, Wfc, bfc):
    return pl.pallas_call(
        _final_body,
        out_shape=jax.ShapeDtypeStruct((_G, Wfc.shape[1]), jnp.float32),
    )(acc, b, batch2d, Wfc, bfc)


def _edge_body(h_hbm, avs_hbm, avd_hbm, src_hbm, dst_hbm, out_hbm,
               avs_v, avd_v, src_v, dst_v, rbuf, sbuf, acc, gsem):
    cid = lax.axis_index("c")
    sid = lax.axis_index("s")
    wid = sid * 2 + cid

    zero16 = jnp.zeros((16,), jnp.float32)

    def zrow(i, _):
        for j in range(9):
            sbuf[i, pl.ds(16 * j, 16)] = zero16
        return 0
    lax.fori_loop(0, _K, zrow, 0)

    zbase = sid * 626
    for t in range(9):
        pltpu.sync_copy(sbuf, acc.at[pl.ds(zbase + _K * t, _K)])
    pltpu.sync_copy(sbuf.at[pl.ds(0, 50)], acc.at[pl.ds(zbase + 576, 50)])

    pltpu.sync_copy(avs_hbm, avs_v)
    pltpu.sync_copy(avd_hbm, avd_v)
    plsc.subcore_barrier()

    iota16 = lax.iota(jnp.int32, 16)

    def chunk_body(c, _):
        ibase = wid * _RPB + c * _CHUNK
        pltpu.sync_copy(src_hbm.at[pl.ds(ibase, _CHUNK)], src_v)
        pltpu.sync_copy(dst_hbm.at[pl.ds(ibase, _CHUNK)], dst_v)

        def batch_body(b, _):
            pltpu.async_copy(h_hbm.at[src_v.at[b]], rbuf, gsem).wait()
            for j in range(_K // 16):
                sl = pl.ds(16 * j, 16)
                isrc = src_v[b, sl]
                idst = jnp.minimum(dst_v[b, sl], _N - 1)
                e = (plsc.load_gather(avs_v, [isrc])
                     + plsc.load_gather(avd_v, [idst]))
                e = jnp.where(e >= 0.0, e, 0.2 * e)
                wv = jnp.exp(e)
                for i in range(16):
                    s = wv[i]
                    row = 16 * j + i
                    for d in range(8):
                        dsl = pl.ds(16 * d, 16)
                        sbuf[row, dsl] = rbuf[row, dsl] * s
                    sbuf[row, pl.ds(128, 16)] = jnp.where(iota16 == 0, s, 0.0)

            pltpu.sync_copy(sbuf, acc.at[dst_v.at[b]], add=True)
            return 0
        lax.fori_loop(0, _CHUNK, batch_body, 0)
        return 0
    lax.fori_loop(0, _NCH, chunk_body, 0)

    plsc.subcore_barrier()
    obase = sid * 625
    pltpu.sync_copy(acc.at[pl.ds(obase, 625)],
                    out_hbm.at[cid].at[pl.ds(obase, 625)])


def _edge_pass(h, avs, avd, src2d, dst2d):
    mesh = plsc.VectorSubcoreMesh(core_axis_name="c", subcore_axis_name="s")
    f = pl.kernel(
        _edge_body,
        out_type=jax.ShapeDtypeStruct((2, _N, _ACCW), jnp.float32),
        mesh=mesh,
        compiler_params=pltpu.CompilerParams(use_tc_tiling_on_sc=False,
                                             needs_layout_passes=False),
        scratch_types=[
            pltpu.VMEM((_N,), jnp.float32),
            pltpu.VMEM((_N,), jnp.float32),
            pltpu.VMEM((_CHUNK, _K), jnp.int32),
            pltpu.VMEM((_CHUNK, _K), jnp.int32),
            pltpu.VMEM((_K, 128), jnp.float32),
            pltpu.VMEM((_K, _ACCW), jnp.float32),
            pltpu.VMEM_SHARED((_ACC_ROWS, _ACCW), jnp.float32),
            pltpu.SemaphoreType.DMA,
        ],
    )
    return f(h, avs, avd, src2d, dst2d)


def kernel(x, edge_index, batch, W1, as1, ad1, b1, W2, as2, ad2, b2, Wfc, bfc):
    src = edge_index[0]
    dst = edge_index[1]
    pad = _EPAD - _E
    srcp = jnp.concatenate(
        [src, jnp.zeros((pad,), jnp.int32)]).reshape(_NW * _RPB, _K)
    dstp = jnp.concatenate(
        [dst, jnp.full((pad,), _N, jnp.int32)]).reshape(_NW * _RPB, _K)

    A1 = jnp.stack([as1, ad1], axis=1)
    A2 = jnp.stack([as2, ad2], axis=1)

    h1, av1 = _dense(x, W1, A1)
    acc1 = _edge_pass(h1, av1[:, 0], av1[:, 1], srcp, dstp)
    h2, av2 = _mid(acc1, b1.reshape(1, _H), W2, A2)
    acc2 = _edge_pass(h2, av2[:, 0], av2[:, 1], srcp, dstp)
    return _final(acc2, b2.reshape(1, _H), batch.reshape(_N, 1), Wfc, bfc)


# speedup vs baseline: 19.5804x; 1.3357x over previous
import functools

import jax
import jax.numpy as jnp
from jax import lax
from jax.experimental import pallas as pl
from jax.experimental.pallas import tpu as pltpu
from jax.experimental.pallas import tpu_sc as plsc

_N = 10000
_E = 320000
_H = 128
_G = 64
_ACCW = 144
_NW = 32
_K = 64
_CHUNK = 8
_NCH = 20
_RPB = _NCH * _CHUNK
_EPAD = _NW * _RPB * _K
_ACC_ROWS = 10016


def _dense_body(x_ref, w_ref, a_ref, p_ref, hb_ref, av_ref):
    h = jnp.dot(x_ref[...], w_ref[...], preferred_element_type=jnp.float32)
    hb_ref[...] = jnp.dot(h, p_ref[...],
                          preferred_element_type=jnp.float32).astype(
                              jnp.bfloat16)
    av_ref[...] = jnp.dot(h, a_ref[...], preferred_element_type=jnp.float32)


def _dense(x, W, A, P):
    return pl.pallas_call(
        _dense_body,
        out_shape=[
            jax.ShapeDtypeStruct((_N, _H), jnp.bfloat16),
            jax.ShapeDtypeStruct((_N, 2), jnp.float32),
        ],
    )(x, W, A, P)


def _mid_body(acc_ref, b_ref, w_ref, a_ref, p_ref, hb_ref, av_ref):
    num = acc_ref[0, :, :128] + acc_ref[1, :, :128]
    den = acc_ref[0, :, 128:129] + acc_ref[1, :, 128:129]
    xr = jnp.maximum(num / (den + 1e-16) + b_ref[...], 0.0)
    h = jnp.dot(xr, w_ref[...], preferred_element_type=jnp.float32)
    hb_ref[...] = jnp.dot(h, p_ref[...],
                          preferred_element_type=jnp.float32).astype(
                              jnp.bfloat16)
    av_ref[...] = jnp.dot(h, a_ref[...], preferred_element_type=jnp.float32)


def _mid(acc, b, W, A, P):
    return pl.pallas_call(
        _mid_body,
        out_shape=[
            jax.ShapeDtypeStruct((_N, _H), jnp.bfloat16),
            jax.ShapeDtypeStruct((_N, 2), jnp.float32),
        ],
    )(acc, b, W, A, P)


def _final_body(acc_ref, b_ref, batch_ref, wfc_ref, bfc_ref, out_ref):
    num = acc_ref[0, :, :128] + acc_ref[1, :, :128]
    den = acc_ref[0, :, 128:129] + acc_ref[1, :, 128:129]
    hf = num / (den + 1e-16) + b_ref[...]
    bt = batch_ref[...]
    rows = []
    for g in range(_G):
        m = bt == g
        rows.append(jnp.max(jnp.where(m, hf, -jnp.inf), axis=0, keepdims=True))
    pooled = jnp.concatenate(rows, axis=0)
    pooled = jnp.where(jnp.isfinite(pooled), pooled, 0.0)
    logits = jnp.dot(pooled, wfc_ref[...],
                     preferred_element_type=jnp.float32) + bfc_ref[...]
    mx = jnp.max(logits, axis=1, keepdims=True)
    s = logits - mx
    lse = jnp.log(jnp.sum(jnp.exp(s), axis=1, keepdims=True))
    out_ref[...] = s - lse


def _final(acc, b, batch2d, Wfc, bfc):
    return pl.pallas_call(
        _final_body,
        out_shape=jax.ShapeDtypeStruct((_G, Wfc.shape[1]), jnp.float32),
    )(acc, b, batch2d, Wfc, bfc)


def _edge_body(h_hbm, avs_hbm, avd_hbm, src_hbm, dst_hbm, out_hbm,
               avs_v, avd_v, src_v, dst_v, rbuf, sbuf, acc, gsem):
    cid = lax.axis_index("c")
    sid = lax.axis_index("s")
    wid = sid * 2 + cid

    zero16 = jnp.zeros((16,), jnp.float32)

    def zrow(i, _):
        for j in range(9):
            sbuf[i, pl.ds(16 * j, 16)] = zero16
        return 0
    lax.fori_loop(0, _K, zrow, 0)

    zbase = sid * 626
    for t in range(9):
        pltpu.sync_copy(sbuf, acc.at[pl.ds(zbase + _K * t, _K)])
    pltpu.sync_copy(sbuf.at[pl.ds(0, 50)], acc.at[pl.ds(zbase + 576, 50)])

    pltpu.sync_copy(avs_hbm, avs_v)
    pltpu.sync_copy(avd_hbm, avd_v)
    plsc.subcore_barrier()

    iota16 = lax.iota(jnp.int32, 16)

    def chunk_body(c, _):
        ibase = wid * _RPB + c * _CHUNK
        pltpu.sync_copy(src_hbm.at[pl.ds(ibase, _CHUNK)], src_v)
        pltpu.sync_copy(dst_hbm.at[pl.ds(ibase, _CHUNK)], dst_v)

        def batch_body(b, _):
            pltpu.async_copy(h_hbm.at[src_v.at[b]], rbuf, gsem).wait()
            for j in range(_K // 16):
                sl = pl.ds(16 * j, 16)
                isrc = src_v[b, sl]
                idst = jnp.minimum(dst_v[b, sl], _N - 1)
                e = (plsc.load_gather(avs_v, [isrc])
                     + plsc.load_gather(avd_v, [idst]))
                e = jnp.where(e >= 0.0, e, 0.2 * e)
                wv = jnp.exp(e)
                for i in range(16):
                    s = wv[i]
                    row = 16 * j + i
                    for d in range(4):
                        ab = rbuf[row, pl.ds(32 * d, 32)]
                        fa, fb = plsc.unpack(
                            ab, format=plsc.PackFormat.INTERLEAVED)
                        sbuf[row, pl.ds(32 * d, 16)] = fa * s
                        sbuf[row, pl.ds(32 * d + 16, 16)] = fb * s
                    sbuf[row, pl.ds(128, 16)] = jnp.where(iota16 == 0, s, 0.0)

            pltpu.sync_copy(sbuf, acc.at[dst_v.at[b]], add=True)
            return 0
        lax.fori_loop(0, _CHUNK, batch_body, 0)
        return 0
    lax.fori_loop(0, _NCH, chunk_body, 0)

    plsc.subcore_barrier()
    obase = sid * 625
    pltpu.sync_copy(acc.at[pl.ds(obase, 625)],
                    out_hbm.at[cid].at[pl.ds(obase, 625)])


def _edge_pass(h, avs, avd, src2d, dst2d):
    mesh = plsc.VectorSubcoreMesh(core_axis_name="c", subcore_axis_name="s")
    f = pl.kernel(
        _edge_body,
        out_type=jax.ShapeDtypeStruct((2, _N, _ACCW), jnp.float32),
        mesh=mesh,
        compiler_params=pltpu.CompilerParams(use_tc_tiling_on_sc=False,
                                             needs_layout_passes=False),
        scratch_types=[
            pltpu.VMEM((_N,), jnp.float32),
            pltpu.VMEM((_N,), jnp.float32),
            pltpu.VMEM((_CHUNK, _K), jnp.int32),
            pltpu.VMEM((_CHUNK, _K), jnp.int32),
            pltpu.VMEM((_K, 128), jnp.bfloat16),
            pltpu.VMEM((_K, _ACCW), jnp.float32),
            pltpu.VMEM_SHARED((_ACC_ROWS, _ACCW), jnp.float32),
            pltpu.SemaphoreType.DMA,
        ],
    )
    return f(h, avs, avd, src2d, dst2d)


def kernel(x, edge_index, batch, W1, as1, ad1, b1, W2, as2, ad2, b2, Wfc, bfc):
    src = edge_index[0]
    dst = edge_index[1]
    pad = _EPAD - _E
    srcp = jnp.concatenate(
        [src, jnp.zeros((pad,), jnp.int32)]).reshape(_NW * _RPB, _K)
    dstp = jnp.concatenate(
        [dst, jnp.full((pad,), _N, jnp.int32)]).reshape(_NW * _RPB, _K)

    A1 = jnp.stack([as1, ad1], axis=1)
    A2 = jnp.stack([as2, ad2], axis=1)

    q = jnp.arange(_H, dtype=jnp.int32)
    perm = 32 * (q // 32) + 16 * (q % 2) + (q % 32) // 2
    P = (perm[:, None] == jnp.arange(_H, dtype=jnp.int32)[None, :]
         ).astype(jnp.float32).T

    h1, av1 = _dense(x, W1, A1, P)
    acc1 = _edge_pass(h1, av1[:, 0], av1[:, 1], srcp, dstp)
    h2, av2 = _mid(acc1, b1.reshape(1, _H), W2, A2, P)
    acc2 = _edge_pass(h2, av2[:, 0], av2[:, 1], srcp, dstp)
    return _final(acc2, b2.reshape(1, _H), batch.reshape(_N, 1), Wfc, bfc)


# speedup vs baseline: 27.4135x; 1.4001x over previous
import functools

import jax
import jax.numpy as jnp
from jax import lax
from jax.experimental import pallas as pl
from jax.experimental.pallas import tpu as pltpu
from jax.experimental.pallas import tpu_sc as plsc

_N = 10000
_E = 320000
_H = 128
_G = 64
_ACCW = 144
_NW = 32
_K = 80
_CHUNK = 5
_NCH = 25
_RPB = _NCH * _CHUNK
_ACC_ROWS = 10016


def _dense_body(x_ref, w_ref, a_ref, p_ref, hb_ref, av_ref):
    h = jnp.dot(x_ref[...], w_ref[...], preferred_element_type=jnp.float32)
    hb_ref[...] = jnp.dot(h, p_ref[...],
                          preferred_element_type=jnp.float32).astype(
                              jnp.bfloat16)
    av_ref[...] = jnp.dot(h, a_ref[...], preferred_element_type=jnp.float32)


def _dense(x, W, A, P):
    return pl.pallas_call(
        _dense_body,
        out_shape=[
            jax.ShapeDtypeStruct((_N, _H), jnp.bfloat16),
            jax.ShapeDtypeStruct((_N, 2), jnp.float32),
        ],
    )(x, W, A, P)


def _mid_body(acc_ref, b_ref, w_ref, a_ref, p_ref, hb_ref, av_ref):
    num = acc_ref[0, :, :128] + acc_ref[1, :, :128]
    den = acc_ref[0, :, 128:129] + acc_ref[1, :, 128:129]
    xr = jnp.maximum(num / (den + 1e-16) + b_ref[...], 0.0)
    h = jnp.dot(xr, w_ref[...], preferred_element_type=jnp.float32)
    hb_ref[...] = jnp.dot(h, p_ref[...],
                          preferred_element_type=jnp.float32).astype(
                              jnp.bfloat16)
    av_ref[...] = jnp.dot(h, a_ref[...], preferred_element_type=jnp.float32)


def _mid(acc, b, W, A, P):
    return pl.pallas_call(
        _mid_body,
        out_shape=[
            jax.ShapeDtypeStruct((_N, _H), jnp.bfloat16),
            jax.ShapeDtypeStruct((_N, 2), jnp.float32),
        ],
    )(acc, b, W, A, P)


def _final_body(acc_ref, b_ref, batch_ref, wfc_ref, bfc_ref, out_ref):
    num = acc_ref[0, :, :128] + acc_ref[1, :, :128]
    den = acc_ref[0, :, 128:129] + acc_ref[1, :, 128:129]
    hf = num / (den + 1e-16) + b_ref[...]
    bt = batch_ref[...]
    rows = []
    for g in range(_G):
        m = bt == g
        rows.append(jnp.max(jnp.where(m, hf, -jnp.inf), axis=0, keepdims=True))
    pooled = jnp.concatenate(rows, axis=0)
    pooled = jnp.where(jnp.isfinite(pooled), pooled, 0.0)
    logits = jnp.dot(pooled, wfc_ref[...],
                     preferred_element_type=jnp.float32) + bfc_ref[...]
    mx = jnp.max(logits, axis=1, keepdims=True)
    s = logits - mx
    lse = jnp.log(jnp.sum(jnp.exp(s), axis=1, keepdims=True))
    out_ref[...] = s - lse


def _final(acc, b, batch2d, Wfc, bfc):
    return pl.pallas_call(
        _final_body,
        out_shape=jax.ShapeDtypeStruct((_G, Wfc.shape[1]), jnp.float32),
    )(acc, b, batch2d, Wfc, bfc)


def _edge_body(h_hbm, avs_hbm, avd_hbm, src_hbm, dst_hbm, out_hbm,
               avs_v, avd_v, src_v, dst_v, rbuf, sbuf, acc, gsem):
    cid = lax.axis_index("c")
    sid = lax.axis_index("s")
    wid = sid * 2 + cid

    zero16 = jnp.zeros((16,), jnp.float32)

    def zrow(i, _):
        for j in range(9):
            sbuf[i, pl.ds(16 * j, 16)] = zero16
        return 0
    lax.fori_loop(0, _K, zrow, 0)

    zbase = sid * 626
    for t in range(7):
        pltpu.sync_copy(sbuf, acc.at[pl.ds(zbase + _K * t, _K)])
    pltpu.sync_copy(sbuf.at[pl.ds(0, 66)], acc.at[pl.ds(zbase + 560, 66)])

    pltpu.sync_copy(avs_hbm, avs_v)
    pltpu.sync_copy(avd_hbm, avd_v)
    plsc.subcore_barrier()

    iota16 = lax.iota(jnp.int32, 16)

    def chunk_body(c, _):
        ibase = wid * _RPB + c * _CHUNK
        pltpu.sync_copy(src_hbm.at[pl.ds(ibase, _CHUNK)], src_v)
        pltpu.sync_copy(dst_hbm.at[pl.ds(ibase, _CHUNK)], dst_v)

        def batch_body(b, _):
            pltpu.async_copy(h_hbm.at[src_v.at[b]], rbuf, gsem).wait()
            for j in range(_K // 16):
                sl = pl.ds(16 * j, 16)
                isrc = src_v[b, sl]
                idst = dst_v[b, sl]
                e = (plsc.load_gather(avs_v, [isrc])
                     + plsc.load_gather(avd_v, [idst]))
                e = jnp.where(e >= 0.0, e, 0.2 * e)
                wv = jnp.exp(e)
                for i in range(16):
                    s = wv[i]
                    row = 16 * j + i
                    for d in range(4):
                        ab = rbuf[row, pl.ds(32 * d, 32)]
                        fa, fb = plsc.unpack(
                            ab, format=plsc.PackFormat.INTERLEAVED)
                        sbuf[row, pl.ds(32 * d, 16)] = fa * s
                        sbuf[row, pl.ds(32 * d + 16, 16)] = fb * s
                    sbuf[row, pl.ds(128, 16)] = jnp.where(iota16 == 0, s, 0.0)

            pltpu.sync_copy(sbuf, acc.at[dst_v.at[b]], add=True)
            return 0
        lax.fori_loop(0, _CHUNK, batch_body, 0)
        return 0
    lax.fori_loop(0, _NCH, chunk_body, 0)

    plsc.subcore_barrier()
    obase = sid * 625
    pltpu.sync_copy(acc.at[pl.ds(obase, 625)],
                    out_hbm.at[cid].at[pl.ds(obase, 625)])


def _edge_pass(h, avs, avd, src2d, dst2d):
    mesh = plsc.VectorSubcoreMesh(core_axis_name="c", subcore_axis_name="s")
    f = pl.kernel(
        _edge_body,
        out_type=jax.ShapeDtypeStruct((2, _N, _ACCW), jnp.float32),
        mesh=mesh,
        compiler_params=pltpu.CompilerParams(use_tc_tiling_on_sc=False,
                                             needs_layout_passes=False),
        scratch_types=[
            pltpu.VMEM((_N,), jnp.float32),
            pltpu.VMEM((_N,), jnp.float32),
            pltpu.VMEM((_CHUNK, _K), jnp.int32),
            pltpu.VMEM((_CHUNK, _K), jnp.int32),
            pltpu.VMEM((_K, 128), jnp.bfloat16),
            pltpu.VMEM((_K, _ACCW), jnp.float32),
            pltpu.VMEM_SHARED((_ACC_ROWS, _ACCW), jnp.float32),
            pltpu.SemaphoreType.DMA,
        ],
    )
    return f(h, avs, avd, src2d, dst2d)


def kernel(x, edge_index, batch, W1, as1, ad1, b1, W2, as2, ad2, b2, Wfc, bfc):
    srcp = edge_index[0].reshape(_NW * _RPB, _K)
    dstp = edge_index[1].reshape(_NW * _RPB, _K)

    A1 = jnp.stack([as1, ad1], axis=1)
    A2 = jnp.stack([as2, ad2], axis=1)

    q = jnp.arange(_H, dtype=jnp.int32)
    perm = 32 * (q // 32) + 16 * (q % 2) + (q % 32) // 2
    P = (perm[:, None] == jnp.arange(_H, dtype=jnp.int32)[None, :]
         ).astype(jnp.float32).T

    h1, av1 = _dense(x, W1, A1, P)
    acc1 = _edge_pass(h1, av1[:, 0], av1[:, 1], srcp, dstp)
    h2, av2 = _mid(acc1, b1.reshape(1, _H), W2, A2, P)
    acc2 = _edge_pass(h2, av2[:, 0], av2[:, 1], srcp, dstp)
    return _final(acc2, b2.reshape(1, _H), batch.reshape(_N, 1), Wfc, bfc)


# speedup vs baseline: 27.4369x; 1.0009x over previous
import functools

import jax
import jax.numpy as jnp
from jax import lax
from jax.experimental import pallas as pl
from jax.experimental.pallas import tpu as pltpu
from jax.experimental.pallas import tpu_sc as plsc

_N = 10000
_E = 320000
_H = 128
_G = 64
_ACCW = 144
_NW = 32
_K = 80
_CHUNK = 5
_NCH = 25
_RPB = _NCH * _CHUNK
_ACC_ROWS = 10016


def _dense_body(x_ref, w_ref, a_ref, p_ref, hb_ref, av_ref):
    h = jnp.dot(x_ref[...], w_ref[...], preferred_element_type=jnp.float32)
    hb_ref[...] = jnp.dot(h, p_ref[...],
                          preferred_element_type=jnp.float32).astype(
                              jnp.bfloat16)
    av_ref[...] = jnp.dot(h, a_ref[...], preferred_element_type=jnp.float32)


def _dense(x, W, A, P):
    return pl.pallas_call(
        _dense_body,
        out_shape=[
            jax.ShapeDtypeStruct((_N, _H), jnp.bfloat16),
            jax.ShapeDtypeStruct((_N, 2), jnp.float32),
        ],
    )(x, W, A, P)


def _mid_body(acc_ref, b_ref, w_ref, a_ref, p_ref, hb_ref, av_ref):
    num = acc_ref[0, :, :128] + acc_ref[1, :, :128]
    den = acc_ref[0, :, 128:129] + acc_ref[1, :, 128:129]
    xr = jnp.maximum(num / (den + 1e-16) + b_ref[...], 0.0)
    h = jnp.dot(xr, w_ref[...], preferred_element_type=jnp.float32)
    hb_ref[...] = jnp.dot(h, p_ref[...],
                          preferred_element_type=jnp.float32).astype(
                              jnp.bfloat16)
    av_ref[...] = jnp.dot(h, a_ref[...], preferred_element_type=jnp.float32)


def _mid(acc, b, W, A, P):
    return pl.pallas_call(
        _mid_body,
        out_shape=[
            jax.ShapeDtypeStruct((_N, _H), jnp.bfloat16),
            jax.ShapeDtypeStruct((_N, 2), jnp.float32),
        ],
    )(acc, b, W, A, P)


def _final_body(acc_ref, b_ref, batch_ref, wfc_ref, bfc_ref, out_ref):
    num = acc_ref[0, :, :128] + acc_ref[1, :, :128]
    den = acc_ref[0, :, 128:129] + acc_ref[1, :, 128:129]
    hf = (num / (den + 1e-16) + b_ref[...]).astype(jnp.bfloat16)
    bt = batch_ref[...]
    ninf = jnp.array(-jnp.inf, jnp.bfloat16)
    rows = []
    for g in range(_G):
        m = bt == g
        rows.append(jnp.max(jnp.where(m, hf, ninf), axis=0, keepdims=True))
    pooled = jnp.concatenate(rows, axis=0).astype(jnp.float32)
    pooled = jnp.where(jnp.isfinite(pooled), pooled, 0.0)
    logits = jnp.dot(pooled, wfc_ref[...],
                     preferred_element_type=jnp.float32) + bfc_ref[...]
    mx = jnp.max(logits, axis=1, keepdims=True)
    s = logits - mx
    lse = jnp.log(jnp.sum(jnp.exp(s), axis=1, keepdims=True))
    out_ref[...] = s - lse


def _final(acc, b, batch2d, Wfc, bfc):
    return pl.pallas_call(
        _final_body,
        out_shape=jax.ShapeDtypeStruct((_G, Wfc.shape[1]), jnp.float32),
    )(acc, b, batch2d, Wfc, bfc)


def _edge_body(h_hbm, avs_hbm, avd_hbm, src_hbm, dst_hbm, out_hbm,
               avs_v, avd_v, src_v, dst_v, rbuf, sbuf, acc, gsem):
    cid = lax.axis_index("c")
    sid = lax.axis_index("s")
    wid = sid * 2 + cid

    zero16 = jnp.zeros((16,), jnp.float32)

    def zrow(i, _):
        for j in range(9):
            sbuf[i, pl.ds(16 * j, 16)] = zero16
        return 0
    lax.fori_loop(0, _K, zrow, 0)

    zbase = sid * 626
    for t in range(7):
        pltpu.sync_copy(sbuf, acc.at[pl.ds(zbase + _K * t, _K)])
    pltpu.sync_copy(sbuf.at[pl.ds(0, 66)], acc.at[pl.ds(zbase + 560, 66)])

    pltpu.sync_copy(avs_hbm, avs_v)
    pltpu.sync_copy(avd_hbm, avd_v)
    plsc.subcore_barrier()

    iota16 = lax.iota(jnp.int32, 16)

    def chunk_body(c, _):
        ibase = wid * _RPB + c * _CHUNK
        pltpu.sync_copy(src_hbm.at[pl.ds(ibase, _CHUNK)], src_v)
        pltpu.sync_copy(dst_hbm.at[pl.ds(ibase, _CHUNK)], dst_v)

        def batch_body(b, _):
            pltpu.async_copy(h_hbm.at[src_v.at[b]], rbuf, gsem).wait()
            for j in range(_K // 16):
                sl = pl.ds(16 * j, 16)
                isrc = src_v[b, sl]
                idst = dst_v[b, sl]
                e = (plsc.load_gather(avs_v, [isrc])
                     + plsc.load_gather(avd_v, [idst]))
                e = jnp.where(e >= 0.0, e, 0.2 * e)
                wv = jnp.exp(e)
                for i in range(16):
                    s = wv[i]
                    row = 16 * j + i
                    for d in range(4):
                        ab = rbuf[row, pl.ds(32 * d, 32)]
                        fa, fb = plsc.unpack(
                            ab, format=plsc.PackFormat.INTERLEAVED)
                        sbuf[row, pl.ds(32 * d, 16)] = fa * s
                        sbuf[row, pl.ds(32 * d + 16, 16)] = fb * s
                    sbuf[row, pl.ds(128, 16)] = jnp.where(iota16 == 0, s, 0.0)

            pltpu.sync_copy(sbuf, acc.at[dst_v.at[b]], add=True)
            return 0
        lax.fori_loop(0, _CHUNK, batch_body, 0)
        return 0
    lax.fori_loop(0, _NCH, chunk_body, 0)

    plsc.subcore_barrier()
    obase = sid * 625
    pltpu.sync_copy(acc.at[pl.ds(obase, 625)],
                    out_hbm.at[cid].at[pl.ds(obase, 625)])


def _edge_pass(h, avs, avd, src2d, dst2d):
    mesh = plsc.VectorSubcoreMesh(core_axis_name="c", subcore_axis_name="s")
    f = pl.kernel(
        _edge_body,
        out_type=jax.ShapeDtypeStruct((2, _N, _ACCW), jnp.float32),
        mesh=mesh,
        compiler_params=pltpu.CompilerParams(use_tc_tiling_on_sc=False,
                                             needs_layout_passes=False),
        scratch_types=[
            pltpu.VMEM((_N,), jnp.float32),
            pltpu.VMEM((_N,), jnp.float32),
            pltpu.VMEM((_CHUNK, _K), jnp.int32),
            pltpu.VMEM((_CHUNK, _K), jnp.int32),
            pltpu.VMEM((_K, 128), jnp.bfloat16),
            pltpu.VMEM((_K, _ACCW), jnp.float32),
            pltpu.VMEM_SHARED((_ACC_ROWS, _ACCW), jnp.float32),
            pltpu.SemaphoreType.DMA,
        ],
    )
    return f(h, avs, avd, src2d, dst2d)


def kernel(x, edge_index, batch, W1, as1, ad1, b1, W2, as2, ad2, b2, Wfc, bfc):
    srcp = edge_index[0].reshape(_NW * _RPB, _K)
    dstp = edge_index[1].reshape(_NW * _RPB, _K)

    A1 = jnp.stack([as1, ad1], axis=1)
    A2 = jnp.stack([as2, ad2], axis=1)

    q = jnp.arange(_H, dtype=jnp.int32)
    perm = 32 * (q // 32) + 16 * (q % 2) + (q % 32) // 2
    P = (perm[:, None] == jnp.arange(_H, dtype=jnp.int32)[None, :]
         ).astype(jnp.float32).T

    h1, av1 = _dense(x, W1, A1, P)
    acc1 = _edge_pass(h1, av1[:, 0], av1[:, 1], srcp, dstp)
    h2, av2 = _mid(acc1, b1.reshape(1, _H), W2, A2, P)
    acc2 = _edge_pass(h2, av2[:, 0], av2[:, 1], srcp, dstp)
    return _final(acc2, b2.reshape(1, _H), batch.reshape(_N, 1), Wfc, bfc)
